# baseline (speedup 1.0000x reference)
import functools

import jax
import jax.numpy as jnp
from jax.experimental import pallas as pl
from jax.experimental.pallas import tpu as pltpu


def _transport_delay_kernel(x_hbm, adj_ref, dist_ref, out_ref,
                            xbuf, lhs_ref, sem,
                            *, B, T, W, t_base):
    wspm_mean = 2.5
    wspm_scale = 1.8
    max_delay_hours = 24.0
    wind_w = 4
    wind_speed_idx = 10

    b = pl.program_id(0)

    def slab_copy(batch, slot):
        return pltpu.make_async_copy(
            x_hbm.at[batch, pl.ds(t_base, W)], xbuf.at[slot], sem.at[slot])

    @pl.when(b == 0)
    def _prologue():
        slab_copy(0, 0).start()

    @pl.when(b + 1 < B)
    def _prefetch():
        slab_copy(b + 1, (b + 1) % 2).start()

    slot = jax.lax.rem(b, 2)
    slab_copy(b, slot).wait()

    xs = xbuf[slot]
    adj = adj_ref[0]
    dist = dist_ref[...]
    N = adj.shape[0]

    wind = xs[W - wind_w:, :, wind_speed_idx]
    wspm_raw = jnp.clip(jnp.mean(wind, axis=0) * wspm_scale + wspm_mean,
                        0.0, None)
    speed_kmh = wspm_raw * 3.6 + 0.001

    tau = jnp.clip(dist / speed_kmh[None, :], 0.0, max_delay_hours)
    t_query = float(T - 1) - tau

    for t in range(W):
        t_abs = float(t_base + t)
        w = jnp.maximum(1.0 - jnp.abs(t_query - t_abs), 0.0)
        lhs_ref[:, t * N:(t + 1) * N] = adj * w

    rhs = xs.reshape(W * N, xs.shape[-1])
    out_ref[0] = jnp.dot(lhs_ref[...], rhs, preferred_element_type=jnp.float32)


def kernel(x_raw, adj, dist_km):
    B, T, N, F = x_raw.shape
    W = 25
    t_base = T - W

    grid = (B,)
    return pl.pallas_call(
        functools.partial(_transport_delay_kernel, B=B, T=T, W=W,
                          t_base=t_base),
        grid=grid,
        in_specs=[
            pl.BlockSpec(memory_space=pl.ANY),
            pl.BlockSpec((1, N, N), lambda b: (b, 0, 0)),
            pl.BlockSpec((N, N), lambda b: (0, 0)),
        ],
        out_specs=pl.BlockSpec((1, N, F), lambda b: (b, 0, 0)),
        out_shape=jax.ShapeDtypeStruct((B, N, F), jnp.float32),
        scratch_shapes=[
            pltpu.VMEM((2, W, N, F), jnp.float32),
            pltpu.VMEM((N, W * N), jnp.float32),
            pltpu.SemaphoreType.DMA((2,)),
        ],
    )(x_raw, adj, dist_km)


# speedup vs baseline: 3.5204x; 3.5204x over previous
import functools

import jax
import jax.numpy as jnp
from jax.experimental import pallas as pl
from jax.experimental.pallas import tpu as pltpu


def _transport_delay_kernel(xs_ref, wind_ref, adj_ref, dist_ref, out_ref,
                            lhs_ref, *, T, W, t_base):
    wspm_mean = 2.5
    wspm_scale = 1.8
    max_delay_hours = 24.0

    adj = adj_ref[0]
    dist = dist_ref[...]
    N = adj.shape[0]

    wspm_raw = jnp.clip(jnp.mean(wind_ref[0], axis=0) * wspm_scale + wspm_mean,
                        0.0, None)
    speed_kmh = wspm_raw * 3.6 + 0.001

    tau = jnp.clip(dist / speed_kmh[None, :], 0.0, max_delay_hours)
    t_query = float(T - 1) - tau

    for t in range(W):
        t_abs = float(t_base + t)
        w = jnp.maximum(1.0 - jnp.abs(t_query - t_abs), 0.0)
        lhs_ref[:, t * N:(t + 1) * N] = (adj * w).astype(jnp.bfloat16)

    out_ref[0] = jnp.dot(lhs_ref[...], xs_ref[0],
                         preferred_element_type=jnp.float32)


def kernel(x_raw, adj, dist_km):
    B, T, N, F = x_raw.shape
    W = 25
    wind_w = 4
    wind_speed_idx = 10
    t_base = T - W

    xs = jax.lax.slice(x_raw, (0, t_base, 0, 0), (B, T, N, F))
    xs = xs.astype(jnp.bfloat16).reshape(B, W * N, F)
    wind = jax.lax.slice(
        x_raw, (0, T - wind_w, 0, wind_speed_idx),
        (B, T, N, wind_speed_idx + 1)).reshape(B, wind_w, N)

    grid = (B,)
    return pl.pallas_call(
        functools.partial(_transport_delay_kernel, T=T, W=W, t_base=t_base),
        grid=grid,
        in_specs=[
            pl.BlockSpec((1, W * N, F), lambda b: (b, 0, 0)),
            pl.BlockSpec((1, wind_w, N), lambda b: (b, 0, 0)),
            pl.BlockSpec((1, N, N), lambda b: (b, 0, 0)),
            pl.BlockSpec((N, N), lambda b: (0, 0)),
        ],
        out_specs=pl.BlockSpec((1, N, F), lambda b: (b, 0, 0)),
        out_shape=jax.ShapeDtypeStruct((B, N, F), jnp.float32),
        scratch_shapes=[pltpu.VMEM((N, W * N), jnp.bfloat16)],
    )(xs, wind, adj, dist_km)


# speedup vs baseline: 5.5066x; 1.5642x over previous
import functools

import jax
import jax.numpy as jnp
from jax.experimental import pallas as pl
from jax.experimental.pallas import tpu as pltpu


def _transport_delay_kernel(xsT_ref, windT_ref, adjT_ref, dist_ref, outT_ref,
                            lhsT_ref, *, T, W, t_base):
    wspm_mean = 2.5
    wspm_scale = 1.8
    max_delay_hours = 24.0

    adjT = adjT_ref[0]
    dist = dist_ref[...]
    N = adjT.shape[0]

    wspm_raw = jnp.clip(
        jnp.mean(windT_ref[0], axis=1, keepdims=True) * wspm_scale + wspm_mean,
        0.0, None)
    speed_kmh = wspm_raw * 3.6 + 0.001

    tau = jnp.clip(dist / speed_kmh, 0.0, max_delay_hours)
    t_query = float(T - 1) - tau

    for t in range(W):
        t_abs = float(t_base + t)
        w = jnp.maximum(1.0 - jnp.abs(t_query - t_abs), 0.0)
        lhsT_ref[t * N:(t + 1) * N, :] = (adjT * w).astype(jnp.bfloat16)

    outT_ref[0] = jnp.dot(xsT_ref[0], lhsT_ref[...],
                          preferred_element_type=jnp.float32)


def kernel(x_raw, adj, dist_km):
    B, T, N, F = x_raw.shape
    W = 25
    wind_w = 4
    wind_speed_idx = 10
    t_base = T - W

    xs = jax.lax.slice(x_raw, (0, t_base, 0, 0), (B, T, N, F))
    xsT = jnp.transpose(xs, (0, 3, 1, 2)).astype(jnp.bfloat16)
    xsT = xsT.reshape(B, F, W * N)
    windT = jnp.transpose(
        jax.lax.slice(x_raw, (0, T - wind_w, 0, wind_speed_idx),
                      (B, T, N, wind_speed_idx + 1)).reshape(B, wind_w, N),
        (0, 2, 1))
    adjT = jnp.swapaxes(adj, 1, 2)

    grid = (B,)
    outT = pl.pallas_call(
        functools.partial(_transport_delay_kernel, T=T, W=W, t_base=t_base),
        grid=grid,
        in_specs=[
            pl.BlockSpec((1, F, W * N), lambda b: (b, 0, 0)),
            pl.BlockSpec((1, N, wind_w), lambda b: (b, 0, 0)),
            pl.BlockSpec((1, N, N), lambda b: (b, 0, 0)),
            pl.BlockSpec((N, N), lambda b: (0, 0)),
        ],
        out_specs=pl.BlockSpec((1, F, N), lambda b: (b, 0, 0)),
        out_shape=jax.ShapeDtypeStruct((B, F, N), jnp.float32),
        scratch_shapes=[pltpu.VMEM((W * N, N), jnp.bfloat16)],
    )(xsT, windT, adjT, dist_km)
    return jnp.swapaxes(outT, 1, 2)


# speedup vs baseline: 6.2643x; 1.1376x over previous
import functools

import jax
import jax.numpy as jnp
from jax.experimental import pallas as pl
from jax.experimental.pallas import tpu as pltpu


def _transport_delay_kernel(xsT_ref, wind_ref, adj_ref, dist_ref, outT_ref,
                            lhs_ref, *, T, W, t_base):
    wspm_mean = 2.5
    wspm_scale = 1.8
    max_delay_hours = 24.0

    adj = adj_ref[0]
    dist = dist_ref[...]
    N = adj.shape[0]

    wspm_raw = jnp.clip(
        jnp.mean(wind_ref[0], axis=0, keepdims=True) * wspm_scale + wspm_mean,
        0.0, None)
    speed_kmh = wspm_raw * 3.6 + 0.001

    tau = jnp.clip(dist / speed_kmh, 0.0, max_delay_hours)
    t_query = float(T - 1) - tau

    for t in range(W):
        t_abs = float(t_base + t)
        w = jnp.maximum(1.0 - jnp.abs(t_query - t_abs), 0.0)
        lhs_ref[:, t * N:(t + 1) * N] = (adj * w).astype(jnp.bfloat16)

    outT_ref[0] = jax.lax.dot_general(
        xsT_ref[0], lhs_ref[...],
        dimension_numbers=(((1,), (1,)), ((), ())),
        preferred_element_type=jnp.float32)


def kernel(x_raw, adj, dist_km):
    B, T, N, F = x_raw.shape
    W = 25
    wind_w = 4
    wind_speed_idx = 10
    t_base = T - W

    xs = jax.lax.slice(x_raw, (0, t_base, 0, 0), (B, T, N, F))
    xsT = jnp.transpose(xs, (0, 3, 1, 2)).astype(jnp.bfloat16)
    xsT = xsT.reshape(B, F, W * N)
    wind = jax.lax.slice(x_raw, (0, T - wind_w, 0, wind_speed_idx),
                         (B, T, N, wind_speed_idx + 1)).reshape(B, wind_w, N)

    grid = (B,)
    outT = pl.pallas_call(
        functools.partial(_transport_delay_kernel, T=T, W=W, t_base=t_base),
        grid=grid,
        in_specs=[
            pl.BlockSpec((1, F, W * N), lambda b: (b, 0, 0)),
            pl.BlockSpec((1, wind_w, N), lambda b: (b, 0, 0)),
            pl.BlockSpec((1, N, N), lambda b: (b, 0, 0)),
            pl.BlockSpec((N, N), lambda b: (0, 0)),
        ],
        out_specs=pl.BlockSpec((1, F, N), lambda b: (b, 0, 0)),
        out_shape=jax.ShapeDtypeStruct((B, F, N), jnp.float32),
        scratch_shapes=[pltpu.VMEM((N, W * N), jnp.bfloat16)],
    )(xsT, wind, adj, dist_km)
    return jnp.swapaxes(outT, 1, 2)


# speedup vs baseline: 8.1667x; 1.3037x over previous
import functools

import jax
import jax.numpy as jnp
from jax.experimental import pallas as pl
from jax.experimental.pallas import tpu as pltpu


def _transport_delay_kernel(xsT_ref, wind_ref, adj_ref, dist_ref, outT_ref,
                            lhs_ref, *, T, W, t_base, PB):
    wspm_mean = 2.5
    wspm_scale = 1.8
    max_delay_hours = 24.0

    dist = dist_ref[...]

    for s in range(PB):
        adj = adj_ref[s]
        N = adj.shape[0]

        wspm_raw = jnp.clip(
            jnp.mean(wind_ref[s], axis=0, keepdims=True) * wspm_scale
            + wspm_mean, 0.0, None)
        speed_kmh = wspm_raw * 3.6 + 0.001

        tau = jnp.clip(dist / speed_kmh, 0.0, max_delay_hours)
        t_query = float(T - 1) - tau

        for t in range(W):
            t_abs = float(t_base + t)
            w = jnp.maximum(1.0 - jnp.abs(t_query - t_abs), 0.0)
            lhs_ref[:, t * N:(t + 1) * N] = (adj * w).astype(jnp.bfloat16)

        outT_ref[s] = jax.lax.dot_general(
            xsT_ref[s], lhs_ref[...],
            dimension_numbers=(((1,), (1,)), ((), ())),
            preferred_element_type=jnp.float32)


def kernel(x_raw, adj, dist_km):
    B, T, N, F = x_raw.shape
    W = 25
    wind_w = 4
    wind_speed_idx = 10
    t_base = T - W

    xs = jax.lax.slice(x_raw, (0, t_base, 0, 0), (B, T, N, F))
    xsT = jnp.transpose(xs, (0, 3, 1, 2)).astype(jnp.bfloat16)
    xsT = xsT.reshape(B, F, W * N)
    wind = jax.lax.slice(x_raw, (0, T - wind_w, 0, wind_speed_idx),
                         (B, T, N, wind_speed_idx + 1)).reshape(B, wind_w, N)

    PB = 2
    grid = (B // PB,)
    outT = pl.pallas_call(
        functools.partial(_transport_delay_kernel, T=T, W=W, t_base=t_base,
                          PB=PB),
        grid=grid,
        in_specs=[
            pl.BlockSpec((PB, F, W * N), lambda b: (b, 0, 0)),
            pl.BlockSpec((PB, wind_w, N), lambda b: (b, 0, 0)),
            pl.BlockSpec((PB, N, N), lambda b: (b, 0, 0)),
            pl.BlockSpec((N, N), lambda b: (0, 0)),
        ],
        out_specs=pl.BlockSpec((PB, F, N), lambda b: (b, 0, 0)),
        out_shape=jax.ShapeDtypeStruct((B, F, N), jnp.float32),
        scratch_shapes=[pltpu.VMEM((N, W * N), jnp.bfloat16)],
    )(xsT, wind, adj, dist_km)
    return jnp.swapaxes(outT, 1, 2)
